# baseline (speedup 1.0000x reference)
import functools

import jax
import jax.numpy as jnp
import numpy as np
from jax import lax
from jax.experimental import pallas as pl
from jax.experimental.pallas import tpu as pltpu
from jax.experimental.pallas import tpu_sc as plsc

TOPK = 2
_INV_SQRT2 = 0.7071067811865476


def _gelu(x):
    return 0.5 * x * (1.0 + lax.erf(x * _INV_SQRT2))


_NOISE_CACHE = {}


def _noise_const(T, E):
    if (T, E) not in _NOISE_CACHE:
        with jax.ensure_compile_time_eval():
            _NOISE_CACHE[(T, E)] = np.asarray(
                jax.random.normal(jax.random.key(42), (T, E), jnp.float32))
    return _NOISE_CACHE[(T, E)]


def _top2(noisy, E):
    TB = noisy.shape[0]
    iota = lax.broadcasted_iota(jnp.int32, (TB, E), 1)
    m1 = jnp.max(noisy, axis=-1, keepdims=True)
    e1 = jnp.min(jnp.where(noisy == m1, iota, E), axis=-1, keepdims=True)
    noisy_m = jnp.where(iota == e1, -jnp.inf, noisy)
    m2 = jnp.max(noisy_m, axis=-1, keepdims=True)
    e2 = jnp.min(jnp.where(noisy_m == m2, iota, E), axis=-1, keepdims=True)
    t = jnp.exp(m2 - m1)
    denom = 1.0 + t
    g1 = 1.0 / denom
    g2 = t / denom
    sel1 = (iota == e1).astype(jnp.float32)
    sel2 = (iota == e2).astype(jnp.float32)
    return sel1, sel2, g1, g2


def _router_pos_body(x_ref, wr_ref, br_ref, wn_ref, bn_ref, nz_ref,
                     pos_ref, g2_ref, be_ref, valid_ref,
                     noisy_s, bc_s, start_s,
                     *, E, TB, NTB, BT, NBE):
    p = pl.program_id(0)
    b = pl.program_id(1)

    @pl.when(p == 0)
    def _phase0():
        x = x_ref[...]
        logits = jnp.dot(x, wr_ref[...], preferred_element_type=jnp.float32) + br_ref[...]
        nl = jnp.dot(x, wn_ref[...], preferred_element_type=jnp.float32) + bn_ref[...]
        noisy = logits + nz_ref[...] * jax.nn.softplus(nl)
        noisy_s[pl.ds(b * TB, TB), :] = noisy
        sel1, sel2, _, _ = _top2(noisy, E)
        bc_s[pl.ds(b, 1), :] = jnp.sum(sel1 + sel2, axis=0, keepdims=True)

    @pl.when(jnp.logical_and(p == 1, b == 0))
    def _offsets():
        bc = bc_s[...]
        counts = jnp.sum(bc, axis=0, keepdims=True)
        padded = jnp.floor((counts + (BT - 1)) / BT) * BT
        ior = lax.broadcasted_iota(jnp.int32, (E, E), 0)
        ioc = lax.broadcasted_iota(jnp.int32, (E, E), 1)
        mexc = (ior < ioc).astype(jnp.float32)
        padstart = jnp.dot(padded, mexc, preferred_element_type=jnp.float32)
        ibr = lax.broadcasted_iota(jnp.int32, (NTB, NTB), 0)
        ibc = lax.broadcasted_iota(jnp.int32, (NTB, NTB), 1)
        aexc = (ibc < ibr).astype(jnp.float32)
        blockstart = jnp.dot(aexc, bc, preferred_element_type=jnp.float32)
        start_s[...] = padstart + blockstart
        jblk = lax.broadcasted_iota(jnp.int32, (1, NBE), 1).astype(jnp.float32)
        acc = jnp.zeros((1, NBE), jnp.float32)
        for e in range(E):
            acc += (jblk >= padstart[0, e] / BT).astype(jnp.float32)
        be_ref[...] = jnp.maximum(acc - 1.0, 0.0).astype(jnp.int32)
        total_blocks = (padstart[0, E - 1] + padded[0, E - 1]) / BT
        valid_ref[...] = (jblk < total_blocks).astype(jnp.int32)

    @pl.when(p == 1)
    def _phase1():
        noisy = noisy_s[pl.ds(b * TB, TB), :]
        sel1, sel2, g1, g2 = _top2(noisy, E)
        itr = lax.broadcasted_iota(jnp.int32, (TB, TB), 0)
        itc = lax.broadcasted_iota(jnp.int32, (TB, TB), 1)
        aexc = (itc < itr).astype(jnp.float32)
        rank = jnp.dot(aexc, sel1 + sel2, preferred_element_type=jnp.float32)
        posmat = start_s[pl.ds(b, 1), :] + rank
        pos1 = jnp.sum(posmat * sel1, axis=1, keepdims=True)
        pos2 = jnp.sum(posmat * sel2, axis=1, keepdims=True)
        pos_ref[...] = jnp.concatenate([pos1, pos2], axis=1).astype(jnp.int32)
        g2_ref[...] = jnp.concatenate([g1, g2], axis=1)


def _router_pos(flat, Wr, br, Wn, bn, noise, *, E, BT, NBE):
    T, d = flat.shape
    TB = 512
    NTB = T // TB
    return pl.pallas_call(
        functools.partial(_router_pos_body, E=E, TB=TB, NTB=NTB, BT=BT, NBE=NBE),
        grid=(2, NTB),
        in_specs=[
            pl.BlockSpec((TB, d), lambda p, b: (jnp.where(p == 0, b, 0), 0)),
            pl.BlockSpec((d, E), lambda p, b: (0, 0)),
            pl.BlockSpec((1, E), lambda p, b: (0, 0)),
            pl.BlockSpec((d, E), lambda p, b: (0, 0)),
            pl.BlockSpec((1, E), lambda p, b: (0, 0)),
            pl.BlockSpec((TB, E), lambda p, b: (b, 0)),
        ],
        out_specs=[
            pl.BlockSpec((TB, TOPK), lambda p, b: (b, 0)),
            pl.BlockSpec((TB, TOPK), lambda p, b: (b, 0)),
            pl.BlockSpec((1, NBE), lambda p, b: (0, 0)),
            pl.BlockSpec((1, NBE), lambda p, b: (0, 0)),
        ],
        out_shape=[
            jax.ShapeDtypeStruct((T, TOPK), jnp.int32),
            jax.ShapeDtypeStruct((T, TOPK), jnp.float32),
            jax.ShapeDtypeStruct((1, NBE), jnp.int32),
            jax.ShapeDtypeStruct((1, NBE), jnp.int32),
        ],
        scratch_shapes=[
            pltpu.VMEM((T, E), jnp.float32),
            pltpu.VMEM((NTB, E), jnp.float32),
            pltpu.VMEM((NTB, E), jnp.float32),
        ],
    )(flat, Wr, br.reshape(1, E), Wn, bn.reshape(1, E), noise)


_SC_MESH = dict(core_axis_name="c", subcore_axis_name="s")
_NW = 32


def _wid():
    return lax.axis_index("s") * 2 + lax.axis_index("c")


def _sc_dispatch(x, p1r, p2r, pad_total):
    _, C, CH = p1r.shape
    T, d = x.shape

    @functools.partial(
        pl.kernel,
        mesh=plsc.VectorSubcoreMesh(**_SC_MESH),
        out_type=jax.ShapeDtypeStruct((pad_total, d), jnp.float32),
        scratch_types=[
            pltpu.VMEM((C, CH), jnp.int32),
            pltpu.VMEM((C, CH), jnp.int32),
            pltpu.VMEM((CH, d), jnp.float32),
            pltpu.SemaphoreType.DMA,
        ],
    )
    def k(x_hbm, p1_hbm, p2_hbm, xs_hbm, idx1_v, idx2_v, buf, sem):
        w = _wid()
        tbase = w * (C * CH)
        pltpu.sync_copy(p1_hbm.at[w], idx1_v)
        pltpu.sync_copy(p2_hbm.at[w], idx2_v)

        def chunk(c, carry):
            pltpu.async_copy(
                x_hbm.at[pl.ds(tbase + c * CH, CH)], buf, sem).wait()
            pltpu.async_copy(buf, xs_hbm.at[idx1_v.at[c]], sem).wait()
            pltpu.async_copy(buf, xs_hbm.at[idx2_v.at[c]], sem).wait()
            return carry

        lax.fori_loop(0, C, chunk, 0)

    return k(x, p1r, p2r)


def _sc_unsort(p13, p23, y):
    C = p13.shape[1]
    T = _NW * C * 32
    d = y.shape[1]

    @functools.partial(
        pl.kernel,
        mesh=plsc.VectorSubcoreMesh(**_SC_MESH),
        out_type=[
            jax.ShapeDtypeStruct((T, d), jnp.float32),
            jax.ShapeDtypeStruct((T, d), jnp.float32),
        ],
        scratch_types=[
            pltpu.VMEM((C, 32), jnp.int32),
            pltpu.VMEM((C, 32), jnp.int32),
            pltpu.VMEM((32, d), jnp.float32),
            pltpu.SemaphoreType.DMA,
        ],
    )
    def k(p1_hbm, p2_hbm, y_hbm, ya_hbm, yb_hbm, idx1_v, idx2_v, rows_v, sem):
        w = _wid()
        base = w * (C * 32)
        pltpu.sync_copy(p1_hbm.at[w], idx1_v)
        pltpu.sync_copy(p2_hbm.at[w], idx2_v)

        def chunk(c, carry):
            pltpu.async_copy(y_hbm.at[idx1_v.at[c]], rows_v, sem).wait()
            pltpu.sync_copy(rows_v, ya_hbm.at[pl.ds(base + c * 32, 32)])
            pltpu.async_copy(y_hbm.at[idx2_v.at[c]], rows_v, sem).wait()
            pltpu.sync_copy(rows_v, yb_hbm.at[pl.ds(base + c * 32, 32)])
            return carry

        lax.fori_loop(0, C, chunk, 0)

    return k(p13, p23, y)


def _ffn_body(be_ref, valid_ref, xs_ref, w1_ref, b1_ref, w2_ref, b2_ref,
              y_ref, w1bf_s, w2bf_s, xbf_s, yacc_s, *, NB, NN):
    b = pl.program_id(0)
    n = pl.program_id(1)
    changed = jnp.logical_or(
        b == 0, be_ref[0, b] != be_ref[0, jnp.maximum(b - 1, 0)])

    @pl.when(changed)
    def _():
        w1bf_s[:, pl.ds(n * NB, NB)] = w1_ref[0].astype(jnp.bfloat16)
        w2bf_s[pl.ds(n * NB, NB), :] = w2_ref[0].astype(jnp.bfloat16)

    @pl.when(valid_ref[0, b] != 0)
    def _():
        @pl.when(n == 0)
        def _():
            xbf_s[...] = xs_ref[...].astype(jnp.bfloat16)

        h = jnp.dot(xbf_s[...], w1bf_s[:, pl.ds(n * NB, NB)],
                    preferred_element_type=jnp.float32)
        h = _gelu(h + b1_ref[0]).astype(jnp.bfloat16)
        contrib = jnp.dot(h, w2bf_s[pl.ds(n * NB, NB), :],
                          preferred_element_type=jnp.float32)

        @pl.when(n == 0)
        def _():
            yacc_s[...] = contrib

        @pl.when(n > 0)
        def _():
            yacc_s[...] += contrib

        @pl.when(n == NN - 1)
        def _():
            y_ref[...] = yacc_s[...] + b2_ref[0]


def _ffn_grouped(xs, W1, b1, W2, b2, block_expert, valid, *, BT):
    P, d = xs.shape
    E, _, H = W1.shape
    NB = min(1024, H)
    NN = H // NB
    NBLK = P // BT

    def _chg(b, be):
        return jnp.logical_or(
            b == 0, be[0, b] != be[0, jnp.maximum(b - 1, 0)])

    grid_spec = pltpu.PrefetchScalarGridSpec(
        num_scalar_prefetch=2,
        grid=(NBLK, NN),
        in_specs=[
            pl.BlockSpec((BT, d), lambda b, n, be, vv: (b, 0)),
            pl.BlockSpec(
                (1, d, NB),
                lambda b, n, be, vv: (
                    be[0, b], 0, jnp.where(_chg(b, be), n, NN - 1))),
            pl.BlockSpec((1, 1, NB), lambda b, n, be, vv: (be[0, b], 0, n)),
            pl.BlockSpec(
                (1, NB, d),
                lambda b, n, be, vv: (
                    be[0, b], jnp.where(_chg(b, be), n, NN - 1), 0)),
            pl.BlockSpec((1, 1, d), lambda b, n, be, vv: (be[0, b], 0, 0)),
        ],
        out_specs=pl.BlockSpec((BT, d), lambda b, n, be, vv: (b, 0)),
        scratch_shapes=[
            pltpu.VMEM((d, H), jnp.bfloat16),
            pltpu.VMEM((H, d), jnp.bfloat16),
            pltpu.VMEM((BT, d), jnp.bfloat16),
            pltpu.VMEM((BT, d), jnp.float32),
        ],
    )
    return pl.pallas_call(
        functools.partial(_ffn_body, NB=NB, NN=NN),
        grid_spec=grid_spec,
        out_shape=jax.ShapeDtypeStruct((P, d), jnp.float32),
    )(block_expert, valid, xs, W1, b1.reshape(E, 1, H), W2,
      b2.reshape(E, 1, d))


def _combine_body(ya_ref, yb_ref, g2_ref, o_ref):
    o_ref[...] = (g2_ref[:, 0:1] * ya_ref[...] + g2_ref[:, 1:2] * yb_ref[...])


def _combine(ya, yb, g2):
    T, d = ya.shape
    TB = 512
    return pl.pallas_call(
        _combine_body,
        grid=(T // TB,),
        in_specs=[
            pl.BlockSpec((TB, d), lambda b: (b, 0)),
            pl.BlockSpec((TB, d), lambda b: (b, 0)),
            pl.BlockSpec((TB, TOPK), lambda b: (b, 0)),
        ],
        out_specs=pl.BlockSpec((TB, d), lambda b: (b, 0)),
        out_shape=jax.ShapeDtypeStruct((T, d), jnp.float32),
    )(ya, yb, g2)


def kernel(x, Wr, br, Wn, bn, W1, b1, W2, b2):
    d = x.shape[-1]
    E = Wr.shape[-1]
    flat = x.reshape(-1, d)
    T = flat.shape[0]
    BT = 256
    PAD_TOTAL = T * TOPK + E * BT
    NBE = 128

    noise = jnp.asarray(_noise_const(T, E))

    pos, g2, block_expert, valid = _router_pos(flat, Wr, br, Wn, bn, noise,
                                               E=E, BT=BT, NBE=NBE)

    p13 = pos[:, 0].reshape(_NW, T // (_NW * 32), 32)
    p23 = pos[:, 1].reshape(_NW, T // (_NW * 32), 32)
    xs = _sc_dispatch(flat, p13, p23, PAD_TOTAL)

    y = _ffn_grouped(xs, W1, b1, W2, b2, block_expert, valid, BT=BT)

    ya, yb = _sc_unsort(p13, p23, y)

    out = _combine(ya, yb, g2)
    return out.reshape(x.shape)


# speedup vs baseline: 1.1393x; 1.1393x over previous
import functools

import jax
import jax.numpy as jnp
import numpy as np
from jax import lax
from jax.experimental import pallas as pl
from jax.experimental.pallas import tpu as pltpu
from jax.experimental.pallas import tpu_sc as plsc

TOPK = 2
_INV_SQRT2 = 0.7071067811865476


def _gelu(x):
    return 0.5 * x * (1.0 + lax.erf(x * _INV_SQRT2))


_NOISE_CACHE = {}


def _noise_const(T, E):
    if (T, E) not in _NOISE_CACHE:
        with jax.ensure_compile_time_eval():
            _NOISE_CACHE[(T, E)] = np.asarray(
                jax.random.normal(jax.random.key(42), (T, E), jnp.float32))
    return _NOISE_CACHE[(T, E)]


def _top2(noisy, E):
    TB = noisy.shape[0]
    iota = lax.broadcasted_iota(jnp.int32, (TB, E), 1)
    m1 = jnp.max(noisy, axis=-1, keepdims=True)
    e1 = jnp.min(jnp.where(noisy == m1, iota, E), axis=-1, keepdims=True)
    noisy_m = jnp.where(iota == e1, -jnp.inf, noisy)
    m2 = jnp.max(noisy_m, axis=-1, keepdims=True)
    e2 = jnp.min(jnp.where(noisy_m == m2, iota, E), axis=-1, keepdims=True)
    t = jnp.exp(m2 - m1)
    denom = 1.0 + t
    g1 = 1.0 / denom
    g2 = t / denom
    sel1 = (iota == e1).astype(jnp.float32)
    sel2 = (iota == e2).astype(jnp.float32)
    return sel1, sel2, g1, g2


def _router_pos_body(x_ref, wr_ref, br_ref, wn_ref, bn_ref, nz_ref,
                     pos_ref, g2_ref, be_ref, valid_ref,
                     noisy_s, bc_s, start_s,
                     *, E, TB, NTB, BT, NBE):
    p = pl.program_id(0)
    b = pl.program_id(1)

    @pl.when(p == 0)
    def _phase0():
        x = x_ref[...]
        logits = jnp.dot(x, wr_ref[...], preferred_element_type=jnp.float32) + br_ref[...]
        nl = jnp.dot(x, wn_ref[...], preferred_element_type=jnp.float32) + bn_ref[...]
        noisy = logits + nz_ref[...] * jax.nn.softplus(nl)
        noisy_s[pl.ds(b * TB, TB), :] = noisy
        sel1, sel2, _, _ = _top2(noisy, E)
        bc_s[pl.ds(b, 1), :] = jnp.sum(sel1 + sel2, axis=0, keepdims=True)

    @pl.when(jnp.logical_and(p == 1, b == 0))
    def _offsets():
        bc = bc_s[...]
        counts = jnp.sum(bc, axis=0, keepdims=True)
        padded = jnp.floor((counts + (BT - 1)) / BT) * BT
        ior = lax.broadcasted_iota(jnp.int32, (E, E), 0)
        ioc = lax.broadcasted_iota(jnp.int32, (E, E), 1)
        mexc = (ior < ioc).astype(jnp.float32)
        padstart = jnp.dot(padded, mexc, preferred_element_type=jnp.float32)
        ibr = lax.broadcasted_iota(jnp.int32, (NTB, NTB), 0)
        ibc = lax.broadcasted_iota(jnp.int32, (NTB, NTB), 1)
        aexc = (ibc < ibr).astype(jnp.float32)
        blockstart = jnp.dot(aexc, bc, preferred_element_type=jnp.float32)
        start_s[...] = padstart + blockstart
        jblk = lax.broadcasted_iota(jnp.int32, (1, NBE), 1).astype(jnp.float32)
        acc = jnp.zeros((1, NBE), jnp.float32)
        for e in range(E):
            acc += (jblk >= padstart[0, e] / BT).astype(jnp.float32)
        be_ref[...] = jnp.maximum(acc - 1.0, 0.0).astype(jnp.int32)
        total_blocks = (padstart[0, E - 1] + padded[0, E - 1]) / BT
        valid_ref[...] = (jblk < total_blocks).astype(jnp.int32)

    @pl.when(p == 1)
    def _phase1():
        noisy = noisy_s[pl.ds(b * TB, TB), :]
        sel1, sel2, g1, g2 = _top2(noisy, E)
        itr = lax.broadcasted_iota(jnp.int32, (TB, TB), 0)
        itc = lax.broadcasted_iota(jnp.int32, (TB, TB), 1)
        aexc = (itc < itr).astype(jnp.float32)
        rank = jnp.dot(aexc, sel1 + sel2, preferred_element_type=jnp.float32)
        posmat = start_s[pl.ds(b, 1), :] + rank
        pos1 = jnp.sum(posmat * sel1, axis=1, keepdims=True)
        pos2 = jnp.sum(posmat * sel2, axis=1, keepdims=True)
        pos_ref[...] = jnp.concatenate([pos1, pos2], axis=1).astype(jnp.int32)
        g2_ref[...] = jnp.concatenate([g1, g2], axis=1)


def _router_pos(flat, Wr, br, Wn, bn, noise, *, E, BT, NBE):
    T, d = flat.shape
    TB = 512
    NTB = T // TB
    return pl.pallas_call(
        functools.partial(_router_pos_body, E=E, TB=TB, NTB=NTB, BT=BT, NBE=NBE),
        grid=(2, NTB),
        in_specs=[
            pl.BlockSpec((TB, d), lambda p, b: (jnp.where(p == 0, b, 0), 0)),
            pl.BlockSpec((d, E), lambda p, b: (0, 0)),
            pl.BlockSpec((1, E), lambda p, b: (0, 0)),
            pl.BlockSpec((d, E), lambda p, b: (0, 0)),
            pl.BlockSpec((1, E), lambda p, b: (0, 0)),
            pl.BlockSpec((TB, E), lambda p, b: (b, 0)),
        ],
        out_specs=[
            pl.BlockSpec((TB, TOPK), lambda p, b: (b, 0)),
            pl.BlockSpec((TB, TOPK), lambda p, b: (b, 0)),
            pl.BlockSpec((1, NBE), lambda p, b: (0, 0)),
            pl.BlockSpec((1, NBE), lambda p, b: (0, 0)),
        ],
        out_shape=[
            jax.ShapeDtypeStruct((T, TOPK), jnp.int32),
            jax.ShapeDtypeStruct((T, TOPK), jnp.float32),
            jax.ShapeDtypeStruct((1, NBE), jnp.int32),
            jax.ShapeDtypeStruct((1, NBE), jnp.int32),
        ],
        scratch_shapes=[
            pltpu.VMEM((T, E), jnp.float32),
            pltpu.VMEM((NTB, E), jnp.float32),
            pltpu.VMEM((NTB, E), jnp.float32),
        ],
    )(flat, Wr, br.reshape(1, E), Wn, bn.reshape(1, E), noise)


_SC_MESH = dict(core_axis_name="c", subcore_axis_name="s")
_NW = 32


def _wid():
    return lax.axis_index("s") * 2 + lax.axis_index("c")


def _sc_dispatch(x, p1r, p2r, pad_total):
    _, C, CH = p1r.shape
    T, d = x.shape

    @functools.partial(
        pl.kernel,
        mesh=plsc.VectorSubcoreMesh(**_SC_MESH),
        out_type=jax.ShapeDtypeStruct((pad_total, d), jnp.float32),
        scratch_types=[
            pltpu.VMEM((C, CH), jnp.int32),
            pltpu.VMEM((C, CH), jnp.int32),
            pltpu.VMEM((CH, d), jnp.float32),
            pltpu.SemaphoreType.DMA,
        ],
    )
    def k(x_hbm, p1_hbm, p2_hbm, xs_hbm, idx1_v, idx2_v, buf, sem):
        w = _wid()
        tbase = w * (C * CH)
        pltpu.sync_copy(p1_hbm.at[w], idx1_v)
        pltpu.sync_copy(p2_hbm.at[w], idx2_v)

        def chunk(c, carry):
            pltpu.async_copy(
                x_hbm.at[pl.ds(tbase + c * CH, CH)], buf, sem).wait()
            pltpu.async_copy(buf, xs_hbm.at[idx1_v.at[c]], sem).wait()
            pltpu.async_copy(buf, xs_hbm.at[idx2_v.at[c]], sem).wait()
            return carry

        lax.fori_loop(0, C, chunk, 0)

    return k(x, p1r, p2r)


def _sc_unsort(p13, p23, y):
    C = p13.shape[1]
    T = _NW * C * 32
    d = y.shape[1]

    @functools.partial(
        pl.kernel,
        mesh=plsc.VectorSubcoreMesh(**_SC_MESH),
        out_type=[
            jax.ShapeDtypeStruct((T, d), jnp.float32),
            jax.ShapeDtypeStruct((T, d), jnp.float32),
        ],
        scratch_types=[
            pltpu.VMEM((C, 32), jnp.int32),
            pltpu.VMEM((C, 32), jnp.int32),
            pltpu.VMEM((32, d), jnp.float32),
            pltpu.SemaphoreType.DMA,
        ],
    )
    def k(p1_hbm, p2_hbm, y_hbm, ya_hbm, yb_hbm, idx1_v, idx2_v, rows_v, sem):
        w = _wid()
        base = w * (C * 32)
        pltpu.sync_copy(p1_hbm.at[w], idx1_v)
        pltpu.sync_copy(p2_hbm.at[w], idx2_v)

        def chunk(c, carry):
            pltpu.async_copy(y_hbm.at[idx1_v.at[c]], rows_v, sem).wait()
            pltpu.sync_copy(rows_v, ya_hbm.at[pl.ds(base + c * 32, 32)])
            pltpu.async_copy(y_hbm.at[idx2_v.at[c]], rows_v, sem).wait()
            pltpu.sync_copy(rows_v, yb_hbm.at[pl.ds(base + c * 32, 32)])
            return carry

        lax.fori_loop(0, C, chunk, 0)

    return k(p13, p23, y)


def _ffn_body(be_ref, valid_ref, xs_ref, w1_ref, b1_ref, w2_ref, b2_ref,
              y_ref, w1bf_s, w2bf_s, xbf_s, yacc_s, *, NB, NN):
    b = pl.program_id(0)
    n = pl.program_id(1)
    changed = jnp.logical_or(
        b == 0, be_ref[0, b] != be_ref[0, jnp.maximum(b - 1, 0)])

    @pl.when(changed)
    def _():
        w1bf_s[:, pl.ds(n * NB, NB)] = w1_ref[0].astype(jnp.bfloat16)
        w2bf_s[pl.ds(n * NB, NB), :] = w2_ref[0].astype(jnp.bfloat16)

    @pl.when(valid_ref[0, b] != 0)
    def _():
        @pl.when(n == 0)
        def _():
            xbf_s[...] = xs_ref[...].astype(jnp.bfloat16)

        h = jnp.dot(xbf_s[...], w1bf_s[:, pl.ds(n * NB, NB)],
                    preferred_element_type=jnp.float32)
        h = _gelu(h + b1_ref[0]).astype(jnp.bfloat16)
        contrib = jnp.dot(h, w2bf_s[pl.ds(n * NB, NB), :],
                          preferred_element_type=jnp.float32)

        @pl.when(n == 0)
        def _():
            yacc_s[...] = contrib

        @pl.when(n > 0)
        def _():
            yacc_s[...] += contrib

        @pl.when(n == NN - 1)
        def _():
            y_ref[...] = yacc_s[...] + b2_ref[0]


def _ffn_grouped(xs, W1, b1, W2, b2, block_expert, valid, *, BT):
    P, d = xs.shape
    E, _, H = W1.shape
    NB = min(1024, H)
    NN = H // NB
    NBLK = P // BT

    def _chg(b, be):
        return jnp.logical_or(
            b == 0, be[0, b] != be[0, jnp.maximum(b - 1, 0)])

    grid_spec = pltpu.PrefetchScalarGridSpec(
        num_scalar_prefetch=2,
        grid=(NBLK, NN),
        in_specs=[
            pl.BlockSpec((BT, d), lambda b, n, be, vv: (b, 0)),
            pl.BlockSpec(
                (1, d, NB),
                lambda b, n, be, vv: (
                    be[0, b], 0, jnp.where(_chg(b, be), n, NN - 1))),
            pl.BlockSpec((1, 1, NB), lambda b, n, be, vv: (be[0, b], 0, n)),
            pl.BlockSpec(
                (1, NB, d),
                lambda b, n, be, vv: (
                    be[0, b], jnp.where(_chg(b, be), n, NN - 1), 0)),
            pl.BlockSpec((1, 1, d), lambda b, n, be, vv: (be[0, b], 0, 0)),
        ],
        out_specs=pl.BlockSpec((BT, d), lambda b, n, be, vv: (b, 0)),
        scratch_shapes=[
            pltpu.VMEM((d, H), jnp.bfloat16),
            pltpu.VMEM((H, d), jnp.bfloat16),
            pltpu.VMEM((BT, d), jnp.bfloat16),
            pltpu.VMEM((BT, d), jnp.float32),
        ],
    )
    return pl.pallas_call(
        functools.partial(_ffn_body, NB=NB, NN=NN),
        grid_spec=grid_spec,
        out_shape=jax.ShapeDtypeStruct((P, d), jnp.float32),
    )(block_expert, valid, xs, W1, b1.reshape(E, 1, H), W2,
      b2.reshape(E, 1, d))


def _combine_body(ya_ref, yb_ref, g2_ref, o_ref):
    o_ref[...] = (g2_ref[:, 0:1] * ya_ref[...] + g2_ref[:, 1:2] * yb_ref[...])


def _combine(ya, yb, g2):
    T, d = ya.shape
    TB = 512
    return pl.pallas_call(
        _combine_body,
        grid=(T // TB,),
        in_specs=[
            pl.BlockSpec((TB, d), lambda b: (b, 0)),
            pl.BlockSpec((TB, d), lambda b: (b, 0)),
            pl.BlockSpec((TB, TOPK), lambda b: (b, 0)),
        ],
        out_specs=pl.BlockSpec((TB, d), lambda b: (b, 0)),
        out_shape=jax.ShapeDtypeStruct((T, d), jnp.float32),
    )(ya, yb, g2)


def kernel(x, Wr, br, Wn, bn, W1, b1, W2, b2):
    d = x.shape[-1]
    E = Wr.shape[-1]
    flat = x.reshape(-1, d)
    T = flat.shape[0]
    BT = 512
    PAD_TOTAL = T * TOPK + E * BT
    NBE = 128

    noise = jnp.asarray(_noise_const(T, E))

    pos, g2, block_expert, valid = _router_pos(flat, Wr, br, Wn, bn, noise,
                                               E=E, BT=BT, NBE=NBE)

    p13 = pos[:, 0].reshape(_NW, T // (_NW * 32), 32)
    p23 = pos[:, 1].reshape(_NW, T // (_NW * 32), 32)
    xs = _sc_dispatch(flat, p13, p23, PAD_TOTAL)

    y = _ffn_grouped(xs, W1, b1, W2, b2, block_expert, valid, BT=BT)

    ya, yb = _sc_unsort(p13, p23, y)

    out = _combine(ya, yb, g2)
    return out.reshape(x.shape)
